# Initial kernel scaffold; baseline (speedup 1.0000x reference)
import jax
import jax.numpy as jnp
from jax.experimental import pallas as pl


def kernel(x, Wr, Wg, Wu, Wd):
    raise NotImplementedError("write your pallas kernel here")


# speedup vs baseline: 1.5563x; 1.5563x over previous
import jax
import jax.numpy as jnp
from jax import lax
from jax.experimental import pallas as pl
from jax.experimental.pallas import tpu as pltpu

_E = 8
_AUX_W = 0.01


def _wq_body(w_ref, q_ref):
    w = w_ref[0]
    s = jnp.maximum(jnp.mean(jnp.abs(w)), 1e-8)
    uq = jnp.clip(jnp.round(w / s), -1.0, 1.0)
    q_ref[0] = (uq * s).astype(jnp.bfloat16)


def _quantize_weights(w):
    e, r, c = w.shape
    return pl.pallas_call(
        _wq_body,
        grid=(e,),
        in_specs=[pl.BlockSpec((1, r, c), lambda i: (i, 0, 0))],
        out_specs=pl.BlockSpec((1, r, c), lambda i: (i, 0, 0)),
        out_shape=jax.ShapeDtypeStruct((e, r, c), jnp.bfloat16),
    )(w)


def _router_body(x_ref, wr_ref, xq_ref, cwt_ref, ps_ref, aux_ref):
    t = pl.program_id(0)
    n_total = pl.num_programs(0) * x_ref.shape[0]
    x = x_ref[...]
    logits = lax.dot_general(x.astype(jnp.bfloat16),
                             wr_ref[...].astype(jnp.bfloat16),
                             (((1,), (1,)), ((), ())),
                             preferred_element_type=jnp.float32)
    mx = jnp.max(logits, axis=1, keepdims=True)
    ex = jnp.exp(logits - mx)
    probs = ex / jnp.sum(ex, axis=1, keepdims=True)

    tn, e = probs.shape
    iota = lax.broadcasted_iota(jnp.int32, (tn, e), 1)
    m1 = jnp.max(probs, axis=1, keepdims=True)
    i1 = jnp.min(jnp.where(probs == m1, iota, e), axis=1, keepdims=True)
    sel1 = iota == i1
    pm = jnp.where(sel1, -1.0, probs)
    m2 = jnp.max(pm, axis=1, keepdims=True)
    i2 = jnp.min(jnp.where(pm == m2, iota, e), axis=1, keepdims=True)
    sel2 = iota == i2
    denom = jnp.maximum(m1 + m2, 1e-9)
    cwt_ref[...] = (m1 / denom) * sel1 + (m2 / denom) * sel2

    amax = jnp.maximum(jnp.max(jnp.abs(x), axis=1, keepdims=True), 1e-8)
    sx = 127.0 / amax
    xq = jnp.clip(jnp.round(x * sx), -127.0, 127.0) / sx
    xq_ref[...] = xq.astype(jnp.bfloat16)

    pcur = jnp.sum(probs, axis=0, keepdims=True)
    fcur = jnp.sum((sel1 | sel2).astype(jnp.float32), axis=0, keepdims=True)
    prev_p = jnp.where(t == 0, 0.0, ps_ref[0:1, :])
    prev_f = jnp.where(t == 0, 0.0, ps_ref[1:2, :])
    newp = prev_p + pcur
    newf = prev_f + fcur
    ps_ref[0:1, :] = newp
    ps_ref[1:2, :] = newf

    @pl.when(t == pl.num_programs(0) - 1)
    def _():
        fp = (newf / n_total) * (newp / n_total)
        aux_ref[...] = (_AUX_W * e) * jnp.sum(fp, keepdims=True).reshape(1, 1)


def _ffn_body(xq_ref, cw_ref, wg_ref, wu_ref, wd_ref, out_ref, acc_ref):
    ei = pl.program_id(0)
    t = pl.program_id(1)
    tn = xq_ref.shape[0]

    cw_all = cw_ref[...]
    eiota = lax.broadcasted_iota(jnp.int32, cw_all.shape, 1)
    cw = jnp.sum(jnp.where(eiota == ei, cw_all, 0.0), axis=1, keepdims=True)

    xq = xq_ref[...]
    g = lax.dot_general(xq, wg_ref[0], (((1,), (1,)), ((), ())),
                        preferred_element_type=jnp.float32)
    u = lax.dot_general(xq, wu_ref[0], (((1,), (1,)), ((), ())),
                        preferred_element_type=jnp.float32)
    m = (g * lax.logistic(g)) * u

    amaxm = jnp.maximum(jnp.max(jnp.abs(m), axis=1, keepdims=True), 1e-8)
    sm = 127.0 / amaxm
    mq = (jnp.clip(jnp.round(m * sm), -127.0, 127.0) / sm).astype(jnp.bfloat16)
    d32 = lax.dot_general(mq, wd_ref[0], (((1,), (1,)), ((), ())),
                          preferred_element_type=jnp.float32)
    y = d32 * cw

    sl = pl.ds(t * tn, tn)
    prev = jnp.where(ei == 0, jnp.zeros_like(y), acc_ref[sl, :])
    newacc = prev + y
    acc_ref[sl, :] = newacc
    out_ref[...] = newacc


def kernel(x, Wr, Wg, Wu, Wd):
    n, d = x.shape
    e, f, _ = Wg.shape

    wgq = _quantize_weights(Wg)
    wuq = _quantize_weights(Wu)
    wdq = _quantize_weights(Wd)

    tn = 256
    nt = n // tn
    xq, cwt, ps, aux2 = pl.pallas_call(
        _router_body,
        grid=(nt,),
        in_specs=[
            pl.BlockSpec((tn, d), lambda t: (t, 0)),
            pl.BlockSpec((e, d), lambda t: (0, 0)),
        ],
        out_specs=[
            pl.BlockSpec((tn, d), lambda t: (t, 0)),
            pl.BlockSpec((tn, e), lambda t: (t, 0)),
            pl.BlockSpec((2, e), lambda t: (0, 0)),
            pl.BlockSpec((1, 1), lambda t: (0, 0)),
        ],
        out_shape=[
            jax.ShapeDtypeStruct((n, d), jnp.bfloat16),
            jax.ShapeDtypeStruct((n, e), jnp.float32),
            jax.ShapeDtypeStruct((2, e), jnp.float32),
            jax.ShapeDtypeStruct((1, 1), jnp.float32),
        ],
    )(x, Wr)

    out = pl.pallas_call(
        _ffn_body,
        grid=(e, nt),
        in_specs=[
            pl.BlockSpec((tn, d), lambda ei, t: (t, 0)),
            pl.BlockSpec((tn, e), lambda ei, t: (t, 0)),
            pl.BlockSpec((1, f, d), lambda ei, t: (ei, 0, 0)),
            pl.BlockSpec((1, f, d), lambda ei, t: (ei, 0, 0)),
            pl.BlockSpec((1, d, f), lambda ei, t: (ei, 0, 0)),
        ],
        out_specs=pl.BlockSpec((tn, d), lambda ei, t: (t, 0)),
        out_shape=jax.ShapeDtypeStruct((n, d), jnp.float32),
        scratch_shapes=[pltpu.VMEM((n, d), jnp.float32)],
        compiler_params=pltpu.CompilerParams(
            vmem_limit_bytes=60 * 1024 * 1024,
        ),
    )(xq, cwt, wgq, wuq, wdq)

    return out, jnp.reshape(aux2, ())
